# Initial kernel scaffold; baseline (speedup 1.0000x reference)
import jax
import jax.numpy as jnp
from jax.experimental import pallas as pl


def kernel(x, edge_index, batch, W1, b1, W2, b2, W3, b3, Wlin, blin):
    raise NotImplementedError("write your pallas kernel here")


# speedup vs baseline: 7.4386x; 7.4386x over previous
import functools

import jax
import jax.numpy as jnp
from jax import lax
from jax.experimental import pallas as pl
from jax.experimental.pallas import tpu as pltpu
from jax.experimental.pallas import tpu_sc as plsc

NCORE = 2
NSUB = 16
NW = NCORE * NSUB
LANE = 128
G = 256


def _mesh():
    return plsc.VectorSubcoreMesh(core_axis_name="c", subcore_axis_name="s",
                                  num_cores=NCORE, num_subcores=NSUB)


@functools.lru_cache(maxsize=None)
def _deg_kernel(n_chunks, nacc):
    cpt = n_chunks // NW
    zr = nacc // NSUB

    def body(dst_hbm, z_hbm, out_hbm, dst_v, e0_v, acc, _sem):
        cid = lax.axis_index("c")
        sid = lax.axis_index("s")
        wid = cid * NSUB + sid
        zbase = pl.multiple_of(sid * zr, 8)
        ebase = pl.multiple_of(wid * cpt, 8)
        pltpu.sync_copy(z_hbm, acc.at[pl.ds(zbase, zr)])
        pltpu.sync_copy(dst_hbm.at[pl.ds(ebase, cpt)], dst_v)
        one_hot = jnp.where(lax.iota(jnp.int32, 16) == 0, 1.0, 0.0)

        def fill(i, carry):
            e0_v[i] = one_hot
            return carry

        lax.fori_loop(0, LANE, fill, 0)
        plsc.subcore_barrier()

        def step(c, carry):
            pltpu.sync_copy(e0_v, acc.at[dst_v.at[c]], add=True)
            return carry

        lax.fori_loop(0, cpt, step, 0)
        plsc.subcore_barrier()
        pltpu.sync_copy(acc.at[pl.ds(zbase, zr)],
                        out_hbm.at[cid, pl.ds(zbase, zr)])

    return pl.kernel(
        body,
        out_type=jax.ShapeDtypeStruct((NCORE, nacc, 16), jnp.float32),
        mesh=_mesh(),
        scratch_types=[
            pltpu.VMEM((cpt, LANE), jnp.int32),
            pltpu.VMEM((LANE, 16), jnp.float32),
            pltpu.VMEM_SHARED((nacc, 16), jnp.float32),
            pltpu.SemaphoreType.DMA,
        ],
    )


@functools.lru_cache(maxsize=None)
def _msg_kernel(n_chunks, nacc, feat):
    cpt = n_chunks // NW
    zr = nacc // NSUB

    def body(g_hbm, src_hbm, dst_hbm, z_hbm, out_hbm,
             src_v, dst_v, rows_v, acc, sem):
        cid = lax.axis_index("c")
        sid = lax.axis_index("s")
        wid = cid * NSUB + sid
        zbase = pl.multiple_of(sid * zr, 8)
        ebase = pl.multiple_of(wid * cpt, 8)
        pltpu.sync_copy(z_hbm, acc.at[pl.ds(zbase, zr)])
        pltpu.sync_copy(src_hbm.at[pl.ds(ebase, cpt)], src_v)
        pltpu.sync_copy(dst_hbm.at[pl.ds(ebase, cpt)], dst_v)
        plsc.subcore_barrier()

        def step(c, carry):
            pltpu.async_copy(g_hbm.at[src_v.at[c]], rows_v, sem).wait()
            pltpu.sync_copy(rows_v, acc.at[dst_v.at[c]], add=True)
            return carry

        lax.fori_loop(0, cpt, step, 0)
        plsc.subcore_barrier()
        pltpu.sync_copy(acc.at[pl.ds(zbase, zr)],
                        out_hbm.at[cid, pl.ds(zbase, zr)])

    return pl.kernel(
        body,
        out_type=jax.ShapeDtypeStruct((NCORE, nacc, feat), jnp.float32),
        mesh=_mesh(),
        scratch_types=[
            pltpu.VMEM((cpt, LANE), jnp.int32),
            pltpu.VMEM((cpt, LANE), jnp.int32),
            pltpu.VMEM((LANE, feat), jnp.float32),
            pltpu.VMEM_SHARED((nacc, feat), jnp.float32),
            pltpu.SemaphoreType.DMA,
        ],
    )


def _dinv(d0, d1):
    return lax.rsqrt(d0 + d1 + 1.0)


def _l1_body(x_ref, w_ref, d0_ref, d1_ref, o_ref):
    dinv = _dinv(d0_ref[...], d1_ref[...])
    o_ref[...] = dinv * jnp.dot(x_ref[...], w_ref[...],
                                preferred_element_type=jnp.float32)


def _l23_body(p_ref, g_ref, d0_ref, d1_ref, b_ref, w_ref, o_ref):
    dinv = _dinv(d0_ref[...], d1_ref[...])
    xl = jnp.maximum(dinv * (p_ref[0] + p_ref[1] + g_ref[...]) + b_ref[...],
                     0.0)
    o_ref[...] = dinv * jnp.dot(xl, w_ref[...],
                                preferred_element_type=jnp.float32)


@functools.lru_cache(maxsize=None)
def _l1_call(npad, feat, bn):
    nb = npad // bn
    return pl.pallas_call(
        _l1_body,
        grid=(nb,),
        in_specs=[
            pl.BlockSpec((bn, feat), lambda i: (i, 0)),
            pl.BlockSpec((feat, feat), lambda i: (0, 0)),
            pl.BlockSpec((bn, 1), lambda i: (i, 0)),
            pl.BlockSpec((bn, 1), lambda i: (i, 0)),
        ],
        out_specs=pl.BlockSpec((bn, feat), lambda i: (i, 0)),
        out_shape=jax.ShapeDtypeStruct((npad, feat), jnp.float32),
    )


@functools.lru_cache(maxsize=None)
def _l23_call(npad, feat, bn):
    nb = npad // bn
    return pl.pallas_call(
        _l23_body,
        grid=(nb,),
        in_specs=[
            pl.BlockSpec((NCORE, bn, feat), lambda i: (0, i, 0)),
            pl.BlockSpec((bn, feat), lambda i: (i, 0)),
            pl.BlockSpec((bn, 1), lambda i: (i, 0)),
            pl.BlockSpec((bn, 1), lambda i: (i, 0)),
            pl.BlockSpec((1, feat), lambda i: (0, 0)),
            pl.BlockSpec((feat, feat), lambda i: (0, 0)),
        ],
        out_specs=pl.BlockSpec((bn, feat), lambda i: (i, 0)),
        out_shape=jax.ShapeDtypeStruct((npad, feat), jnp.float32),
    )


@functools.lru_cache(maxsize=None)
def _final_call(npad, feat, bn, ncls):
    nb = npad // bn

    def body(p_ref, g_ref, d0_ref, d1_ref, b_ref, batch_ref, wl_ref, bl_ref,
             o_ref, sums, cnt):
        i = pl.program_id(0)

        @pl.when(i == 0)
        def _():
            sums[...] = jnp.zeros_like(sums)
            cnt[...] = jnp.zeros_like(cnt)

        dinv = _dinv(d0_ref[...], d1_ref[...])
        h3 = dinv * (p_ref[0] + p_ref[1] + g_ref[...]) + b_ref[...]
        gid = lax.broadcasted_iota(jnp.int32, (G, bn), 0)
        oh = (gid == batch_ref[...]).astype(jnp.float32)
        sums[...] += jnp.dot(oh, h3, preferred_element_type=jnp.float32)
        cnt[...] += jnp.sum(oh, axis=1, keepdims=True)

        @pl.when(i == nb - 1)
        def _():
            pooled = sums[...] / jnp.maximum(cnt[...], 1.0)
            o_ref[...] = jnp.dot(pooled, wl_ref[...],
                                 preferred_element_type=jnp.float32) + bl_ref[...]

    return pl.pallas_call(
        body,
        grid=(nb,),
        in_specs=[
            pl.BlockSpec((NCORE, bn, feat), lambda i: (0, i, 0)),
            pl.BlockSpec((bn, feat), lambda i: (i, 0)),
            pl.BlockSpec((bn, 1), lambda i: (i, 0)),
            pl.BlockSpec((bn, 1), lambda i: (i, 0)),
            pl.BlockSpec((1, feat), lambda i: (0, 0)),
            pl.BlockSpec((1, bn), lambda i: (0, i)),
            pl.BlockSpec((feat, ncls), lambda i: (0, 0)),
            pl.BlockSpec((1, ncls), lambda i: (0, 0)),
        ],
        out_specs=pl.BlockSpec((G, ncls), lambda i: (0, 0)),
        out_shape=jax.ShapeDtypeStruct((G, ncls), jnp.float32),
        scratch_shapes=[
            pltpu.VMEM((G, feat), jnp.float32),
            pltpu.VMEM((G, 1), jnp.float32),
        ],
    )


def kernel(x, edge_index, batch, W1, b1, W2, b2, W3, b3, Wlin, blin):
    n, feat = x.shape
    e = edge_index.shape[1]
    ncls = Wlin.shape[1]
    bn = 512
    npad = ((n + 1023) // 1024) * 1024
    echunk = NW * LANE * 8
    epad = ((e + echunk - 1) // echunk) * echunk
    n_chunks = epad // LANE

    src = jnp.concatenate(
        [edge_index[0], jnp.zeros((epad - e,), jnp.int32)]).reshape(-1, LANE)
    dst = jnp.concatenate(
        [edge_index[1], jnp.full((epad - e,), n, jnp.int32)]).reshape(-1, LANE)
    xp = jnp.concatenate([x, jnp.zeros((npad - n, feat), x.dtype)])
    bp = jnp.concatenate([batch, jnp.full((npad - n,), G, batch.dtype)])
    z_rows = jnp.zeros((npad // NSUB, feat), jnp.float32)
    z16 = jnp.zeros((npad // NSUB, 16), jnp.float32)

    deg = _deg_kernel(n_chunks, npad)(dst, z16)
    d0 = deg[0, :, 0:1]
    d1 = deg[1, :, 0:1]

    g1 = _l1_call(npad, feat, bn)(xp, W1, d0, d1)
    p1 = _msg_kernel(n_chunks, npad, feat)(g1, src, dst, z_rows)
    g2 = _l23_call(npad, feat, bn)(p1, g1, d0, d1, b1.reshape(1, -1), W2)
    p2 = _msg_kernel(n_chunks, npad, feat)(g2, src, dst, z_rows)
    g3 = _l23_call(npad, feat, bn)(p2, g2, d0, d1, b2.reshape(1, -1), W3)
    p3 = _msg_kernel(n_chunks, npad, feat)(g3, src, dst, z_rows)
    out = _final_call(npad, feat, bn, ncls)(
        p3, g3, d0, d1, b3.reshape(1, -1), bp.reshape(1, -1), Wlin,
        blin.reshape(1, -1))
    return out


# speedup vs baseline: 7.7359x; 1.0400x over previous
import functools

import jax
import jax.numpy as jnp
from jax import lax
from jax.experimental import pallas as pl
from jax.experimental.pallas import tpu as pltpu
from jax.experimental.pallas import tpu_sc as plsc

NCORE = 2
NSUB = 16
NW = NCORE * NSUB
LANE = 64
G = 256


def _mesh():
    return plsc.VectorSubcoreMesh(core_axis_name="c", subcore_axis_name="s",
                                  num_cores=NCORE, num_subcores=NSUB)


@functools.lru_cache(maxsize=None)
def _deg_kernel(n_chunks, nacc):
    cpt = n_chunks // NW
    zr = nacc // NSUB

    def body(dst_hbm, z_hbm, out_hbm, dst_v, e0_v, acc, _sem):
        cid = lax.axis_index("c")
        sid = lax.axis_index("s")
        wid = cid * NSUB + sid
        zbase = pl.multiple_of(sid * zr, 8)
        ebase = pl.multiple_of(wid * cpt, 8)
        pltpu.sync_copy(z_hbm, acc.at[pl.ds(zbase, zr)])
        pltpu.sync_copy(dst_hbm.at[pl.ds(ebase, cpt)], dst_v)
        one_hot = jnp.where(lax.iota(jnp.int32, 16) == 0, 1.0, 0.0)

        def fill(i, carry):
            e0_v[i] = one_hot
            return carry

        lax.fori_loop(0, LANE, fill, 0)
        plsc.subcore_barrier()

        def step(c, carry):
            pltpu.sync_copy(e0_v, acc.at[dst_v.at[c]], add=True)
            return carry

        lax.fori_loop(0, cpt, step, 0)
        plsc.subcore_barrier()
        pltpu.sync_copy(acc.at[pl.ds(zbase, zr)],
                        out_hbm.at[cid, pl.ds(zbase, zr)])

    return pl.kernel(
        body,
        out_type=jax.ShapeDtypeStruct((NCORE, nacc, 16), jnp.float32),
        mesh=_mesh(),
        scratch_types=[
            pltpu.VMEM((cpt, LANE), jnp.int32),
            pltpu.VMEM((LANE, 16), jnp.float32),
            pltpu.VMEM_SHARED((nacc, 16), jnp.float32),
            pltpu.SemaphoreType.DMA,
        ],
    )


NB = 2
LA = 1


@functools.lru_cache(maxsize=None)
def _msg_kernel(n_chunks, n, npad, feat):
    cpt = n_chunks // NW
    nacc = ((n + 128) // 128) * 128
    zr = nacc // NSUB
    tail = npad - nacc
    assert cpt >= 2 * NB and zr % 8 == 0 and tail >= 0

    def body(g_hbm, idx_hbm, z_hbm, out_hbm,
             idx_v, rows_v, acc, gsem, ssem):
        cid = lax.axis_index("c")
        sid = lax.axis_index("s")
        wid = cid * NSUB + sid
        zbase = pl.multiple_of(sid * zr, 8)
        ebase = pl.multiple_of(wid * cpt, 8)
        pltpu.sync_copy(z_hbm.at[pl.ds(0, zr)], acc.at[pl.ds(zbase, zr)])
        pltpu.sync_copy(idx_hbm.at[pl.ds(ebase, cpt)], idx_v)
        if tail:
            @pl.when(sid == 0)
            def _():
                pltpu.sync_copy(z_hbm.at[pl.ds(0, tail)],
                                out_hbm.at[cid, pl.ds(nacc, tail)])
        plsc.subcore_barrier()

        def src_at(c):
            return idx_v.at[c, pl.ds(0, LANE)]

        def dst_at(c):
            return idx_v.at[c, pl.ds(LANE, LANE)]

        def gather_start(c, b):
            pltpu.async_copy(g_hbm.at[src_at(c)], rows_v.at[b], gsem.at[b])

        def gather_wait(c, b):
            pltpu.make_async_copy(g_hbm.at[src_at(c)], rows_v.at[b],
                                  gsem.at[b]).wait()

        def scatter_start(c, b):
            pltpu.async_copy(rows_v.at[b], acc.at[dst_at(c)], ssem.at[b],
                             add=True)

        def scatter_wait(c, b):
            pltpu.make_async_copy(rows_v.at[b], acc.at[dst_at(c)],
                                  ssem.at[b]).wait()

        for k in range(LA):
            gather_start(k, k)

        def step(c, carry):
            ca = c + LA
            ba = lax.rem(ca, NB)

            @pl.when(ca < cpt)
            def _():
                @pl.when(ca >= NB)
                def _():
                    scatter_wait(ca - NB, ba)
                gather_start(ca, ba)

            b = lax.rem(c, NB)
            gather_wait(c, b)
            scatter_start(c, b)
            return carry

        lax.fori_loop(0, cpt, step, 0)
        for k in range(NB):
            c = cpt - NB + k
            scatter_wait(c, c % NB)
        plsc.subcore_barrier()
        pltpu.sync_copy(acc.at[pl.ds(zbase, zr)],
                        out_hbm.at[cid, pl.ds(zbase, zr)])

    return pl.kernel(
        body,
        out_type=jax.ShapeDtypeStruct((NCORE, npad, feat), jnp.float32),
        mesh=_mesh(),
        scratch_types=[
            pltpu.VMEM((cpt, 2 * LANE), jnp.int32),
            pltpu.VMEM((NB, LANE, feat), jnp.float32),
            pltpu.VMEM_SHARED((nacc, feat), jnp.float32),
            pltpu.SemaphoreType.DMA((NB,)),
            pltpu.SemaphoreType.DMA((NB,)),
        ],
    )


def _dinv(d0, d1):
    return lax.rsqrt(d0 + d1 + 1.0)


def _l1_body(x_ref, w_ref, d0_ref, d1_ref, o_ref):
    dinv = _dinv(d0_ref[...], d1_ref[...])
    o_ref[...] = dinv * jnp.dot(x_ref[...], w_ref[...],
                                preferred_element_type=jnp.float32)


def _l23_body(p_ref, g_ref, d0_ref, d1_ref, b_ref, w_ref, o_ref):
    dinv = _dinv(d0_ref[...], d1_ref[...])
    xl = jnp.maximum(dinv * (p_ref[0] + p_ref[1] + g_ref[...]) + b_ref[...],
                     0.0)
    o_ref[...] = dinv * jnp.dot(xl, w_ref[...],
                                preferred_element_type=jnp.float32)


@functools.lru_cache(maxsize=None)
def _l1_call(npad, feat, bn):
    nb = npad // bn
    return pl.pallas_call(
        _l1_body,
        grid=(nb,),
        in_specs=[
            pl.BlockSpec((bn, feat), lambda i: (i, 0)),
            pl.BlockSpec((feat, feat), lambda i: (0, 0)),
            pl.BlockSpec((bn, 1), lambda i: (i, 0)),
            pl.BlockSpec((bn, 1), lambda i: (i, 0)),
        ],
        out_specs=pl.BlockSpec((bn, feat), lambda i: (i, 0)),
        out_shape=jax.ShapeDtypeStruct((npad, feat), jnp.float32),
    )


@functools.lru_cache(maxsize=None)
def _l23_call(npad, feat, bn):
    nb = npad // bn
    return pl.pallas_call(
        _l23_body,
        grid=(nb,),
        in_specs=[
            pl.BlockSpec((NCORE, bn, feat), lambda i: (0, i, 0)),
            pl.BlockSpec((bn, feat), lambda i: (i, 0)),
            pl.BlockSpec((bn, 1), lambda i: (i, 0)),
            pl.BlockSpec((bn, 1), lambda i: (i, 0)),
            pl.BlockSpec((1, feat), lambda i: (0, 0)),
            pl.BlockSpec((feat, feat), lambda i: (0, 0)),
        ],
        out_specs=pl.BlockSpec((bn, feat), lambda i: (i, 0)),
        out_shape=jax.ShapeDtypeStruct((npad, feat), jnp.float32),
    )


@functools.lru_cache(maxsize=None)
def _final_call(npad, feat, bn, ncls):
    nb = npad // bn

    def body(p_ref, g_ref, d0_ref, d1_ref, b_ref, batch_ref, wl_ref, bl_ref,
             o_ref, sums, cnt):
        i = pl.program_id(0)

        @pl.when(i == 0)
        def _():
            sums[...] = jnp.zeros_like(sums)
            cnt[...] = jnp.zeros_like(cnt)

        dinv = _dinv(d0_ref[...], d1_ref[...])
        h3 = dinv * (p_ref[0] + p_ref[1] + g_ref[...]) + b_ref[...]
        gid = lax.broadcasted_iota(jnp.int32, (G, bn), 0)
        oh = (gid == batch_ref[...]).astype(jnp.float32)
        sums[...] += jnp.dot(oh, h3, preferred_element_type=jnp.float32)
        cnt[...] += jnp.sum(oh, axis=1, keepdims=True)

        @pl.when(i == nb - 1)
        def _():
            pooled = sums[...] / jnp.maximum(cnt[...], 1.0)
            o_ref[...] = jnp.dot(pooled, wl_ref[...],
                                 preferred_element_type=jnp.float32) + bl_ref[...]

    return pl.pallas_call(
        body,
        grid=(nb,),
        in_specs=[
            pl.BlockSpec((NCORE, bn, feat), lambda i: (0, i, 0)),
            pl.BlockSpec((bn, feat), lambda i: (i, 0)),
            pl.BlockSpec((bn, 1), lambda i: (i, 0)),
            pl.BlockSpec((bn, 1), lambda i: (i, 0)),
            pl.BlockSpec((1, feat), lambda i: (0, 0)),
            pl.BlockSpec((1, bn), lambda i: (0, i)),
            pl.BlockSpec((feat, ncls), lambda i: (0, 0)),
            pl.BlockSpec((1, ncls), lambda i: (0, 0)),
        ],
        out_specs=pl.BlockSpec((G, ncls), lambda i: (0, 0)),
        out_shape=jax.ShapeDtypeStruct((G, ncls), jnp.float32),
        scratch_shapes=[
            pltpu.VMEM((G, feat), jnp.float32),
            pltpu.VMEM((G, 1), jnp.float32),
        ],
    )


def kernel(x, edge_index, batch, W1, b1, W2, b2, W3, b3, Wlin, blin):
    n, feat = x.shape
    e = edge_index.shape[1]
    ncls = Wlin.shape[1]
    bn = 512
    npad = ((n + 1023) // 1024) * 1024
    echunk = NW * LANE * 8
    epad = ((e + echunk - 1) // echunk) * echunk
    n_chunks = epad // LANE

    src = jnp.concatenate(
        [edge_index[0], jnp.zeros((epad - e,), jnp.int32)]).reshape(-1, LANE)
    dst = jnp.concatenate(
        [edge_index[1], jnp.full((epad - e,), n, jnp.int32)]).reshape(-1, LANE)
    idx = jnp.concatenate([src, dst], axis=1)
    xp = jnp.concatenate([x, jnp.zeros((npad - n, feat), x.dtype)])
    bp = jnp.concatenate([batch, jnp.full((npad - n,), G, batch.dtype)])
    z_rows = jnp.zeros((npad // NSUB, feat), jnp.float32)
    z16 = jnp.zeros((npad // NSUB, 16), jnp.float32)

    deg = _deg_kernel(n_chunks, npad)(dst, z16)
    d0 = deg[0, :, 0:1]
    d1 = deg[1, :, 0:1]

    g1 = _l1_call(npad, feat, bn)(xp, W1, d0, d1)
    p1 = _msg_kernel(n_chunks, n, npad, feat)(g1, idx, z_rows)
    g2 = _l23_call(npad, feat, bn)(p1, g1, d0, d1, b1.reshape(1, -1), W2)
    p2 = _msg_kernel(n_chunks, n, npad, feat)(g2, idx, z_rows)
    g3 = _l23_call(npad, feat, bn)(p2, g2, d0, d1, b2.reshape(1, -1), W3)
    p3 = _msg_kernel(n_chunks, n, npad, feat)(g3, idx, z_rows)
    out = _final_call(npad, feat, bn, ncls)(
        p3, g3, d0, d1, b3.reshape(1, -1), bp.reshape(1, -1), Wlin,
        blin.reshape(1, -1))
    return out
